# baseline (speedup 1.0000x reference)
import functools

import jax
import jax.numpy as jnp
from jax import lax
from jax.experimental import pallas as pl
from jax.experimental.pallas import tpu as pltpu
from jax.experimental.pallas import tpu_sc as plsc

_B, _F, _N = 8, 128, 32
_M = _F * _N
_CH = _M // 16
_NGC = 64
_SIF = 16000
_COEF = _NGC / _SIF

_mesh = plsc.VectorSubcoreMesh(
    core_axis_name="c", subcore_axis_name="s", num_cores=2, num_subcores=16
)


@functools.partial(
    pl.kernel,
    out_type=(
        jax.ShapeDtypeStruct((_B * 2 * _M,), jnp.float32),
        jax.ShapeDtypeStruct((_B * 16,), jnp.int32),
    ),
    mesh=_mesh,
    compiler_params=pltpu.CompilerParams(needs_layout_passes=False),
    scratch_types=[
        pltpu.VMEM((2 * _M,), jnp.float32),
        pltpu.VMEM((_M,), jnp.float32),
        pltpu.VMEM((_M,), jnp.float32),
        pltpu.VMEM((_F,), jnp.int32),
        pltpu.VMEM((_M + 16,), jnp.float32),
        pltpu.VMEM((_M + 16,), jnp.float32),
        pltpu.VMEM((_M + 16,), jnp.float32),
        pltpu.VMEM((_M + 16,), jnp.float32),
        pltpu.VMEM((_M + 16,), jnp.float32),
        pltpu.VMEM((_M + 16,), jnp.float32),
        pltpu.VMEM((17,), jnp.float32),
        pltpu.VMEM((2 * _M,), jnp.float32),
        pltpu.VMEM((16,), jnp.int32),
    ],
)
def _resolve_kernel(fr_hbm, off_hbm, out_hbm, ng_hbm,
                    FR, S, E, OFF, Rs, Re, Rsr, Rer, Rr, Rp, SH, OUT, NG):
    wid = lax.axis_index("s") * 2 + lax.axis_index("c")

    @pl.when(wid < _B)
    def _():
        b = wid
        pltpu.sync_copy(fr_hbm.at[pl.ds(b * 2 * _M, 2 * _M)], FR)
        pltpu.sync_copy(off_hbm.at[pl.ds(b * _F, _F)], OFF)

        iota = lax.iota(jnp.int32, 16)
        lane0 = iota == 0
        neg_inf = jnp.float32(jnp.finfo(jnp.float32).min)

        def sort_body(f, carry):
            b0 = f * 32
            g0 = f * 64 + 2 * iota
            ak = plsc.load_gather(FR, [g0])
            av = plsc.load_gather(FR, [g0 + 1])
            bk = plsc.load_gather(FR, [g0 + 32])
            bv = plsc.load_gather(FR, [g0 + 33])
            ak, av = plsc.sort_key_val(ak, av)
            bk, bv = plsc.sort_key_val(bk, bv)
            rbk = lax.rev(bk, (0,))
            rbv = lax.rev(bv, (0,))
            ta = ak <= rbk
            lok = jnp.where(ta, ak, rbk)
            lov = jnp.where(ta, av, rbv)
            hik = jnp.where(ta, rbk, ak)
            hiv = jnp.where(ta, rbv, av)
            lok, lov = plsc.sort_key_val(lok, lov)
            hik, hiv = plsc.sort_key_val(hik, hiv)
            S[pl.ds(b0, 16)] = lok
            S[pl.ds(b0 + 16, 16)] = hik
            E[pl.ds(b0, 16)] = lov
            E[pl.ds(b0 + 16, 16)] = hiv
            return carry

        lax.fori_loop(0, _F, sort_body, 0)

        SH[pl.ds(0, 16)] = jnp.full((16,), neg_inf, jnp.float32)
        coef = jnp.float32(_COEF)

        def scan_body(i, carry):
            c_m, c_s, c_e, c_sr, c_er, c_r, c_g = carry
            sv = S[pl.ds(i * 16, 16)]
            ev = E[pl.ds(i * 16, 16)]
            off = plsc.load_gather(
                OFF, [jnp.zeros((16,), jnp.int32) + (i // 2)]
            ).astype(jnp.float32)
            t1 = ((sv - off) * coef).astype(jnp.int32)
            t2 = ((ev - off) * coef).astype(jnp.int32)
            rv = jnp.where((t1 <= 0) | (t2 >= _NGC - 1),
                           jnp.float32(0), jnp.float32(1))
            cm = plsc.cummax(ev)
            SH[pl.ds(1, 16)] = cm
            shifted = SH[pl.ds(0, 16)]
            excl = jnp.maximum(shifted, c_m)
            flags = sv > excl
            gidx = c_g + plsc.cumsum(flags.astype(jnp.int32)) - 1

            srv = sv * rv
            erv = ev * rv
            for ref, v, c in ((Rs, sv, c_s), (Re, ev, c_e), (Rsr, srv, c_sr),
                              (Rer, erv, c_er), (Rr, rv, c_r)):
                p_incl = c + plsc.cumsum(v)
                plsc.store_scatter(ref, [gidx], p_incl - v, mask=flags)
            pos = (iota + i * 16).astype(jnp.float32)
            plsc.store_scatter(Rp, [gidx], pos, mask=flags)

            return (jnp.maximum(c_m, jnp.max(ev)),
                    c_s + jnp.sum(sv), c_e + jnp.sum(ev),
                    c_sr + jnp.sum(srv), c_er + jnp.sum(erv),
                    c_r + jnp.sum(rv),
                    c_g + plsc.all_reduce_population_count(flags))

        init = (neg_inf, jnp.float32(0), jnp.float32(0), jnp.float32(0),
                jnp.float32(0), jnp.float32(0), jnp.zeros((16,), jnp.int32))
        (_, t_s, t_e, t_sr, t_er, t_r, g_cnt) = lax.fori_loop(
            0, _CH, scan_body, init)

        zf = jnp.zeros((16,), jnp.float32)
        for ref, tot in ((Rs, t_s), (Re, t_e), (Rsr, t_sr), (Rer, t_er),
                         (Rr, t_r), (Rp, jnp.float32(_M))):
            plsc.store_scatter(ref, [g_cnt], zf + tot, mask=lane0)
        NG[...] = g_cnt

        def fin_body(j, carry):
            base = j * 16
            g_i = iota + base
            valid = g_i < g_cnt
            d_s = Rs[pl.ds(base + 1, 16)] - Rs[pl.ds(base, 16)]
            d_e = Re[pl.ds(base + 1, 16)] - Re[pl.ds(base, 16)]
            d_sr = Rsr[pl.ds(base + 1, 16)] - Rsr[pl.ds(base, 16)]
            d_er = Rer[pl.ds(base + 1, 16)] - Rer[pl.ds(base, 16)]
            d_r = Rr[pl.ds(base + 1, 16)] - Rr[pl.ds(base, 16)]
            d_p = Rp[pl.ds(base + 1, 16)] - Rp[pl.ds(base, 16)]
            has1 = d_r > jnp.float32(0.5)
            num_s = jnp.where(has1, d_sr, d_s)
            num_e = jnp.where(has1, d_er, d_e)
            den = jnp.where(has1, d_r, jnp.maximum(d_p, jnp.float32(1)))
            os_ = jnp.where(valid, num_s / den, jnp.float32(0))
            oe_ = jnp.where(valid, num_e / den, jnp.float32(0))
            idx2 = (g_i * 2).astype(jnp.int32)
            plsc.store_scatter(OUT, [idx2], os_)
            plsc.store_scatter(OUT, [idx2 + 1], oe_)
            return carry

        lax.fori_loop(0, _CH, fin_body, 0)

        pltpu.sync_copy(OUT, out_hbm.at[pl.ds(b * 2 * _M, 2 * _M)])
        pltpu.sync_copy(NG, ng_hbm.at[pl.ds(b * 16, 16)])


def kernel(frames_of_fragments_batch, frame_offsets_samples_batch):
    B, F, N, _ = frames_of_fragments_batch.shape
    M = F * N
    fr_flat = frames_of_fragments_batch.reshape(B * M * 2)
    off_flat = frame_offsets_samples_batch.reshape(B * F)
    out_flat, ng_flat = _resolve_kernel(fr_flat, off_flat)
    resolved = out_flat.reshape(B, M, 2)
    num_groups = ng_flat.reshape(B, 16)[:, 0]
    return resolved, num_groups


# speedup vs baseline: 2.2679x; 2.2679x over previous
import functools

import jax
import jax.numpy as jnp
from jax import lax
from jax.experimental import pallas as pl
from jax.experimental.pallas import tpu as pltpu
from jax.experimental.pallas import tpu_sc as plsc

_B, _F, _N = 8, 128, 32
_M = _F * _N
_CH = _M // 16
_NGC = 64
_SIF = 16000
_COEF = _NGC / _SIF

_mesh = plsc.VectorSubcoreMesh(
    core_axis_name="c", subcore_axis_name="s", num_cores=2, num_subcores=16
)


@functools.partial(
    pl.kernel,
    out_type=(
        jax.ShapeDtypeStruct((_B * 2 * _M,), jnp.float32),
        jax.ShapeDtypeStruct((_B * 16,), jnp.int32),
    ),
    mesh=_mesh,
    compiler_params=pltpu.CompilerParams(needs_layout_passes=False),
    scratch_types=[
        pltpu.VMEM((2 * _M,), jnp.float32),
        pltpu.VMEM((_M,), jnp.float32),
        pltpu.VMEM((_M,), jnp.float32),
        pltpu.VMEM((_F,), jnp.int32),
        pltpu.VMEM((_M + 16,), jnp.float32),
        pltpu.VMEM((_M + 16,), jnp.float32),
        pltpu.VMEM((_M + 16,), jnp.float32),
        pltpu.VMEM((_M + 16,), jnp.float32),
        pltpu.VMEM((_M + 16,), jnp.float32),
        pltpu.VMEM((_M + 16,), jnp.float32),
        pltpu.VMEM((17,), jnp.float32),
        pltpu.VMEM((2 * _M,), jnp.float32),
        pltpu.VMEM((16,), jnp.int32),
    ],
)
def _resolve_kernel(fr_hbm, off_hbm, out_hbm, ng_hbm,
                    FR, S, E, OFF, Rs, Re, Rsr, Rer, Rr, Rp, SH, OUT, NG):
    wid = lax.axis_index("s") * 2 + lax.axis_index("c")

    @pl.when(wid < _B)
    def _():
        b = wid
        pltpu.sync_copy(fr_hbm.at[pl.ds(b * 2 * _M, 2 * _M)], FR)
        pltpu.sync_copy(off_hbm.at[pl.ds(b * _F, _F)], OFF)

        iota = lax.iota(jnp.int32, 16)
        lane0 = iota == 0
        neg_inf = jnp.float32(jnp.finfo(jnp.float32).min)

        def tr_body(k, carry):
            bn = k // 128
            fb = (k // 16) % 8
            d = k % 16
            rot = (iota + d) & 15
            src = (bn * 16 + iota) * 256 + fb * 16 + rot
            dst = (fb * 16 + rot) * 32 + bn * 16 + iota
            plsc.store_scatter(S, [dst], plsc.load_gather(FR, [src]))
            plsc.store_scatter(E, [dst], plsc.load_gather(FR, [src + 128]))
            return carry

        lax.fori_loop(0, 256, tr_body, 0)

        def sort_body(f, carry):
            b0 = f * 32
            ak = S[pl.ds(b0, 16)]
            bk = S[pl.ds(b0 + 16, 16)]
            av = E[pl.ds(b0, 16)]
            bv = E[pl.ds(b0 + 16, 16)]
            ak, av = plsc.sort_key_val(ak, av)
            bk, bv = plsc.sort_key_val(bk, bv)
            rbk = lax.rev(bk, (0,))
            rbv = lax.rev(bv, (0,))
            ta = ak <= rbk
            lok = jnp.where(ta, ak, rbk)
            lov = jnp.where(ta, av, rbv)
            hik = jnp.where(ta, rbk, ak)
            hiv = jnp.where(ta, rbv, av)
            lok, lov = plsc.sort_key_val(lok, lov)
            hik, hiv = plsc.sort_key_val(hik, hiv)
            S[pl.ds(b0, 16)] = lok
            S[pl.ds(b0 + 16, 16)] = hik
            E[pl.ds(b0, 16)] = lov
            E[pl.ds(b0 + 16, 16)] = hiv
            return carry

        lax.fori_loop(0, _F, sort_body, 0)

        SH[pl.ds(0, 16)] = jnp.full((16,), neg_inf, jnp.float32)
        coef = jnp.float32(_COEF)

        def scan_body(i, carry):
            c_m, c_s, c_e, c_sr, c_er, c_r, c_g = carry
            sv = S[pl.ds(i * 16, 16)]
            ev = E[pl.ds(i * 16, 16)]
            off = plsc.load_gather(
                OFF, [jnp.zeros((16,), jnp.int32) + (i // 2)]
            ).astype(jnp.float32)
            t1 = ((sv - off) * coef).astype(jnp.int32)
            t2 = ((ev - off) * coef).astype(jnp.int32)
            rv = jnp.where((t1 <= 0) | (t2 >= _NGC - 1),
                           jnp.float32(0), jnp.float32(1))
            cm = plsc.cummax(ev)
            SH[pl.ds(1, 16)] = cm
            shifted = SH[pl.ds(0, 16)]
            excl = jnp.maximum(shifted, c_m)
            flags = sv > excl
            gidx = c_g + plsc.cumsum(flags.astype(jnp.int32)) - 1

            srv = sv * rv
            erv = ev * rv
            for ref, v, c in ((Rs, sv, c_s), (Re, ev, c_e), (Rsr, srv, c_sr),
                              (Rer, erv, c_er), (Rr, rv, c_r)):
                p_incl = c + plsc.cumsum(v)
                plsc.store_scatter(ref, [gidx], p_incl - v, mask=flags)
            pos = (iota + i * 16).astype(jnp.float32)
            plsc.store_scatter(Rp, [gidx], pos, mask=flags)

            return (jnp.maximum(c_m, jnp.max(ev)),
                    c_s + jnp.sum(sv), c_e + jnp.sum(ev),
                    c_sr + jnp.sum(srv), c_er + jnp.sum(erv),
                    c_r + jnp.sum(rv),
                    c_g + plsc.all_reduce_population_count(flags))

        init = (neg_inf, jnp.float32(0), jnp.float32(0), jnp.float32(0),
                jnp.float32(0), jnp.float32(0), jnp.zeros((16,), jnp.int32))
        (_, t_s, t_e, t_sr, t_er, t_r, g_cnt) = lax.fori_loop(
            0, _CH, scan_body, init)

        zf = jnp.zeros((16,), jnp.float32)
        for ref, tot in ((Rs, t_s), (Re, t_e), (Rsr, t_sr), (Rer, t_er),
                         (Rr, t_r), (Rp, jnp.float32(_M))):
            plsc.store_scatter(ref, [g_cnt], zf + tot, mask=lane0)
        NG[...] = g_cnt

        def fin_body(j, carry):
            base = j * 16
            g_i = iota + base
            valid = g_i < g_cnt
            d_s = Rs[pl.ds(base + 1, 16)] - Rs[pl.ds(base, 16)]
            d_e = Re[pl.ds(base + 1, 16)] - Re[pl.ds(base, 16)]
            d_sr = Rsr[pl.ds(base + 1, 16)] - Rsr[pl.ds(base, 16)]
            d_er = Rer[pl.ds(base + 1, 16)] - Rer[pl.ds(base, 16)]
            d_r = Rr[pl.ds(base + 1, 16)] - Rr[pl.ds(base, 16)]
            d_p = Rp[pl.ds(base + 1, 16)] - Rp[pl.ds(base, 16)]
            has1 = d_r > jnp.float32(0.5)
            num_s = jnp.where(has1, d_sr, d_s)
            num_e = jnp.where(has1, d_er, d_e)
            den = jnp.where(has1, d_r, jnp.maximum(d_p, jnp.float32(1)))
            os_ = jnp.where(valid, num_s / den, jnp.float32(0))
            oe_ = jnp.where(valid, num_e / den, jnp.float32(0))
            idx_s = (j >> 3) * 256 + (j & 7) * 16 + iota
            plsc.store_scatter(OUT, [idx_s], os_)
            plsc.store_scatter(OUT, [idx_s + 128], oe_)
            return carry

        lax.fori_loop(0, _CH, fin_body, 0)

        pltpu.sync_copy(OUT, out_hbm.at[pl.ds(b * 2 * _M, 2 * _M)])
        pltpu.sync_copy(NG, ng_hbm.at[pl.ds(b * 16, 16)])


def kernel(frames_of_fragments_batch, frame_offsets_samples_batch):
    B, F, N, _ = frames_of_fragments_batch.shape
    M = F * N
    fr_flat = frames_of_fragments_batch.transpose(0, 2, 3, 1).reshape(B * M * 2)
    off_flat = frame_offsets_samples_batch.reshape(B * F)
    out_flat, ng_flat = _resolve_kernel(fr_flat, off_flat)
    resolved = (out_flat.reshape(B, M // 128, 2, 128)
                .transpose(0, 1, 3, 2).reshape(B, M, 2))
    num_groups = ng_flat.reshape(B, 16)[:, 0]
    return resolved, num_groups


# speedup vs baseline: 2.3356x; 1.0299x over previous
import functools

import jax
import jax.numpy as jnp
from jax import lax
from jax.experimental import pallas as pl
from jax.experimental.pallas import tpu as pltpu
from jax.experimental.pallas import tpu_sc as plsc

_B, _F, _N = 8, 128, 32
_M = _F * _N
_CH = _M // 16
_NGC = 64
_SIF = 16000
_COEF = _NGC / _SIF

_mesh = plsc.VectorSubcoreMesh(
    core_axis_name="c", subcore_axis_name="s", num_cores=2, num_subcores=16
)


@functools.partial(
    pl.kernel,
    out_type=(
        jax.ShapeDtypeStruct((_B * 2 * _M,), jnp.float32),
        jax.ShapeDtypeStruct((_B * 16,), jnp.int32),
    ),
    mesh=_mesh,
    compiler_params=pltpu.CompilerParams(needs_layout_passes=False),
    scratch_types=[
        pltpu.VMEM((2 * _M,), jnp.float32),
        pltpu.VMEM((_M,), jnp.float32),
        pltpu.VMEM((_M,), jnp.float32),
        pltpu.VMEM((_F,), jnp.int32),
        pltpu.VMEM((_M + 16,), jnp.float32),
        pltpu.VMEM((_M + 16,), jnp.float32),
        pltpu.VMEM((_M + 16,), jnp.float32),
        pltpu.VMEM((_M + 16,), jnp.float32),
        pltpu.VMEM((_M + 16,), jnp.float32),
        pltpu.VMEM((_M + 16,), jnp.float32),
        pltpu.VMEM((17,), jnp.float32),
        pltpu.VMEM((2 * _M,), jnp.float32),
        pltpu.VMEM((16,), jnp.int32),
    ],
)
def _resolve_kernel(fr_hbm, off_hbm, out_hbm, ng_hbm,
                    FR, S, E, OFF, Rs, Re, Rsr, Rer, Rr, Rp, SH, OUT, NG):
    wid = lax.axis_index("s") * 2 + lax.axis_index("c")

    @pl.when(wid < _B)
    def _():
        b = wid
        pltpu.sync_copy(fr_hbm.at[pl.ds(b * 2 * _M, 2 * _M)], FR)
        pltpu.sync_copy(off_hbm.at[pl.ds(b * _F, _F)], OFF)

        iota = lax.iota(jnp.int32, 16)
        lane0 = iota == 0
        neg_inf = jnp.float32(jnp.finfo(jnp.float32).min)

        def tr_body(k, carry):
            bn = k // 128
            fb = (k // 16) % 8
            d = k % 16
            rot = (iota + d) & 15
            src = (bn * 16 + iota) * 256 + fb * 16 + rot
            dst = (fb * 16 + rot) * 32 + bn * 16 + iota
            plsc.store_scatter(S, [dst], plsc.load_gather(FR, [src]))
            plsc.store_scatter(E, [dst], plsc.load_gather(FR, [src + 128]))
            return carry

        lax.fori_loop(0, 256, tr_body, 0)

        def sort_body(f, carry):
            b0 = f * 32
            ak = S[pl.ds(b0, 16)]
            bk = S[pl.ds(b0 + 16, 16)]
            av = E[pl.ds(b0, 16)]
            bv = E[pl.ds(b0 + 16, 16)]
            ak, av = plsc.sort_key_val(ak, av)
            bk, bv = plsc.sort_key_val(bk, bv)
            rbk = lax.rev(bk, (0,))
            rbv = lax.rev(bv, (0,))
            ta = ak <= rbk
            lok = jnp.where(ta, ak, rbk)
            lov = jnp.where(ta, av, rbv)
            hik = jnp.where(ta, rbk, ak)
            hiv = jnp.where(ta, rbv, av)
            lok, lov = plsc.sort_key_val(lok, lov)
            hik, hiv = plsc.sort_key_val(hik, hiv)
            S[pl.ds(b0, 16)] = lok
            S[pl.ds(b0 + 16, 16)] = hik
            E[pl.ds(b0, 16)] = lov
            E[pl.ds(b0 + 16, 16)] = hiv
            return carry

        lax.fori_loop(0, _F, sort_body, 0)

        SH[pl.ds(0, 16)] = jnp.full((16,), neg_inf, jnp.float32)
        coef = jnp.float32(_COEF)

        def scan_body(i, carry):
            c_m, c_s, c_e, c_sr, c_er, c_r, c_g = carry
            sv = S[pl.ds(i * 16, 16)]
            ev = E[pl.ds(i * 16, 16)]
            off = plsc.load_gather(
                OFF, [jnp.zeros((16,), jnp.int32) + (i // 2)]
            ).astype(jnp.float32)
            t1 = ((sv - off) * coef).astype(jnp.int32)
            t2 = ((ev - off) * coef).astype(jnp.int32)
            rv = jnp.where((t1 <= 0) | (t2 >= _NGC - 1),
                           jnp.float32(0), jnp.float32(1))
            cm = plsc.cummax(ev)
            lane15 = jnp.full((16,), 15, jnp.int32)
            SH[pl.ds(1, 16)] = cm
            shifted = SH[pl.ds(0, 16)]
            c_m_new = jnp.maximum(c_m, plsc.load_gather(SH, [lane15 + 1]))
            excl = jnp.maximum(shifted, c_m)
            flags = sv > excl
            gidx = c_g + plsc.cumsum(flags.astype(jnp.int32)) - 1

            srv = sv * rv
            erv = ev * rv
            new_c = []
            for ref, v, c in ((Rs, sv, c_s), (Re, ev, c_e), (Rsr, srv, c_sr),
                              (Rer, erv, c_er), (Rr, rv, c_r)):
                p_incl = c + plsc.cumsum(v)
                plsc.store_scatter(ref, [gidx], p_incl - v, mask=flags)
                SH[pl.ds(1, 16)] = p_incl
                new_c.append(plsc.load_gather(SH, [lane15 + 1]))
            pos = (iota + i * 16).astype(jnp.float32)
            plsc.store_scatter(Rp, [gidx], pos, mask=flags)

            return (c_m_new, new_c[0], new_c[1], new_c[2], new_c[3], new_c[4],
                    c_g + plsc.all_reduce_population_count(flags))

        zf16 = jnp.zeros((16,), jnp.float32)
        init = (jnp.full((16,), neg_inf, jnp.float32), zf16, zf16, zf16,
                zf16, zf16, jnp.zeros((16,), jnp.int32))
        (_, t_s, t_e, t_sr, t_er, t_r, g_cnt) = lax.fori_loop(
            0, _CH, scan_body, init)

        zf = jnp.zeros((16,), jnp.float32)
        for ref, tot in ((Rs, t_s), (Re, t_e), (Rsr, t_sr), (Rer, t_er),
                         (Rr, t_r), (Rp, jnp.float32(_M))):
            plsc.store_scatter(ref, [g_cnt], zf + tot, mask=lane0)
        NG[...] = g_cnt

        def zero_body(j, carry):
            OUT[pl.ds(j * 16, 16)] = zf16
            return carry

        lax.fori_loop(0, 2 * _CH, zero_body, 0)
        g_scal = jnp.max(g_cnt)

        def fin_body(j, carry):
            base = j * 16
            g_i = iota + base
            valid = g_i < g_cnt
            d_s = Rs[pl.ds(base + 1, 16)] - Rs[pl.ds(base, 16)]
            d_e = Re[pl.ds(base + 1, 16)] - Re[pl.ds(base, 16)]
            d_sr = Rsr[pl.ds(base + 1, 16)] - Rsr[pl.ds(base, 16)]
            d_er = Rer[pl.ds(base + 1, 16)] - Rer[pl.ds(base, 16)]
            d_r = Rr[pl.ds(base + 1, 16)] - Rr[pl.ds(base, 16)]
            d_p = Rp[pl.ds(base + 1, 16)] - Rp[pl.ds(base, 16)]
            has1 = d_r > jnp.float32(0.5)
            num_s = jnp.where(has1, d_sr, d_s)
            num_e = jnp.where(has1, d_er, d_e)
            den = jnp.where(has1, d_r, jnp.maximum(d_p, jnp.float32(1)))
            os_ = jnp.where(valid, num_s / den, jnp.float32(0))
            oe_ = jnp.where(valid, num_e / den, jnp.float32(0))
            idx_s = (j >> 3) * 256 + (j & 7) * 16 + iota
            plsc.store_scatter(OUT, [idx_s], os_)
            plsc.store_scatter(OUT, [idx_s + 128], oe_)
            return carry

        lax.fori_loop(0, (g_scal + 15) // 16, fin_body, 0)

        pltpu.sync_copy(OUT, out_hbm.at[pl.ds(b * 2 * _M, 2 * _M)])
        pltpu.sync_copy(NG, ng_hbm.at[pl.ds(b * 16, 16)])


def kernel(frames_of_fragments_batch, frame_offsets_samples_batch):
    B, F, N, _ = frames_of_fragments_batch.shape
    M = F * N
    fr_flat = frames_of_fragments_batch.transpose(0, 2, 3, 1).reshape(B * M * 2)
    off_flat = frame_offsets_samples_batch.reshape(B * F)
    out_flat, ng_flat = _resolve_kernel(fr_flat, off_flat)
    resolved = (out_flat.reshape(B, M // 128, 2, 128)
                .transpose(0, 1, 3, 2).reshape(B, M, 2))
    num_groups = ng_flat.reshape(B, 16)[:, 0]
    return resolved, num_groups


# speedup vs baseline: 2.4678x; 1.0566x over previous
import functools

import jax
import jax.numpy as jnp
from jax import lax
from jax.experimental import pallas as pl
from jax.experimental.pallas import tpu as pltpu
from jax.experimental.pallas import tpu_sc as plsc

_B, _F, _N = 8, 128, 32
_M = _F * _N
_CH = _M // 16
_NGC = 64
_SIF = 16000
_COEF = _NGC / _SIF

_mesh = plsc.VectorSubcoreMesh(
    core_axis_name="c", subcore_axis_name="s", num_cores=2, num_subcores=16
)


@functools.partial(
    pl.kernel,
    out_type=(
        jax.ShapeDtypeStruct((_B * 2 * _M,), jnp.float32),
        jax.ShapeDtypeStruct((_B * 16,), jnp.int32),
    ),
    mesh=_mesh,
    compiler_params=pltpu.CompilerParams(needs_layout_passes=False),
    scratch_types=[
        pltpu.VMEM((2 * _M,), jnp.float32),
        pltpu.VMEM((_M,), jnp.float32),
        pltpu.VMEM((_M,), jnp.float32),
        pltpu.VMEM((_F,), jnp.int32),
        pltpu.VMEM((_M + 16,), jnp.float32),
        pltpu.VMEM((_M + 16,), jnp.float32),
        pltpu.VMEM((_M + 16,), jnp.float32),
        pltpu.VMEM((_M + 16,), jnp.float32),
        pltpu.VMEM((_M + 16,), jnp.float32),
        pltpu.VMEM((_M + 16,), jnp.float32),
        pltpu.VMEM((17,), jnp.float32),
        pltpu.VMEM((2 * _M,), jnp.float32),
        pltpu.VMEM((16,), jnp.int32),
    ],
)
def _resolve_kernel(fr_hbm, off_hbm, out_hbm, ng_hbm,
                    FR, S, E, OFF, Rs, Re, Rsr, Rer, Rr, Rp, SH, OUT, NG):
    wid = lax.axis_index("s") * 2 + lax.axis_index("c")

    @pl.when(wid < _B)
    def _():
        b = wid
        pltpu.sync_copy(fr_hbm.at[pl.ds(b * 2 * _M, 2 * _M)], FR)
        pltpu.sync_copy(off_hbm.at[pl.ds(b * _F, _F)], OFF)

        iota = lax.iota(jnp.int32, 16)
        lane0 = iota == 0
        neg_inf = jnp.float32(jnp.finfo(jnp.float32).min)

        def tr_body(k, carry):
            bn = k // 32
            fb = (k // 4) % 8
            d0 = (k % 4) * 4
            for u in range(4):
                rot = (iota + (d0 + u)) & 15
                src = (bn * 16 + iota) * 256 + fb * 16 + rot
                dst = (fb * 16 + rot) * 32 + bn * 16 + iota
                plsc.store_scatter(S, [dst], plsc.load_gather(FR, [src]))
                plsc.store_scatter(E, [dst], plsc.load_gather(FR, [src + 128]))
            return carry

        lax.fori_loop(0, 64, tr_body, 0)

        def sort_body(f, carry):
            b0 = f * 32
            ak = S[pl.ds(b0, 16)]
            bk = S[pl.ds(b0 + 16, 16)]
            av = E[pl.ds(b0, 16)]
            bv = E[pl.ds(b0 + 16, 16)]
            ak, av = plsc.sort_key_val(ak, av)
            bk, bv = plsc.sort_key_val(bk, bv)
            rbk = lax.rev(bk, (0,))
            rbv = lax.rev(bv, (0,))
            ta = ak <= rbk
            lok = jnp.where(ta, ak, rbk)
            lov = jnp.where(ta, av, rbv)
            hik = jnp.where(ta, rbk, ak)
            hiv = jnp.where(ta, rbv, av)
            lok, lov = plsc.sort_key_val(lok, lov)
            hik, hiv = plsc.sort_key_val(hik, hiv)
            S[pl.ds(b0, 16)] = lok
            S[pl.ds(b0 + 16, 16)] = hik
            E[pl.ds(b0, 16)] = lov
            E[pl.ds(b0 + 16, 16)] = hiv
            return carry

        lax.fori_loop(0, _F, sort_body, 0)

        SH[pl.ds(0, 16)] = jnp.full((16,), neg_inf, jnp.float32)
        coef = jnp.float32(_COEF)

        def scan_body(i, carry):
            c_m, c_s, c_e, c_sr, c_er, c_r, c_g = carry
            sv = S[pl.ds(i * 16, 16)]
            ev = E[pl.ds(i * 16, 16)]
            off = plsc.load_gather(
                OFF, [jnp.zeros((16,), jnp.int32) + (i // 2)]
            ).astype(jnp.float32)
            t1 = ((sv - off) * coef).astype(jnp.int32)
            t2 = ((ev - off) * coef).astype(jnp.int32)
            rv = jnp.where((t1 <= 0) | (t2 >= _NGC - 1),
                           jnp.float32(0), jnp.float32(1))
            cm = plsc.cummax(ev)
            lane15 = jnp.full((16,), 15, jnp.int32)
            SH[pl.ds(1, 16)] = cm
            shifted = SH[pl.ds(0, 16)]
            c_m_new = jnp.maximum(c_m, plsc.load_gather(SH, [lane15 + 1]))
            excl = jnp.maximum(shifted, c_m)
            flags = sv > excl
            gidx = c_g + plsc.cumsum(flags.astype(jnp.int32)) - 1

            srv = sv * rv
            erv = ev * rv
            new_c = []
            for ref, v, c in ((Rs, sv, c_s), (Re, ev, c_e), (Rsr, srv, c_sr),
                              (Rer, erv, c_er), (Rr, rv, c_r)):
                p_incl = c + plsc.cumsum(v)
                plsc.store_scatter(ref, [gidx], p_incl - v, mask=flags)
                SH[pl.ds(1, 16)] = p_incl
                new_c.append(plsc.load_gather(SH, [lane15 + 1]))
            pos = (iota + i * 16).astype(jnp.float32)
            plsc.store_scatter(Rp, [gidx], pos, mask=flags)

            return (c_m_new, new_c[0], new_c[1], new_c[2], new_c[3], new_c[4],
                    c_g + plsc.all_reduce_population_count(flags))

        zf16 = jnp.zeros((16,), jnp.float32)
        init = (jnp.full((16,), neg_inf, jnp.float32), zf16, zf16, zf16,
                zf16, zf16, jnp.zeros((16,), jnp.int32))
        (_, t_s, t_e, t_sr, t_er, t_r, g_cnt) = lax.fori_loop(
            0, _CH, scan_body, init)

        zf = jnp.zeros((16,), jnp.float32)
        for ref, tot in ((Rs, t_s), (Re, t_e), (Rsr, t_sr), (Rer, t_er),
                         (Rr, t_r), (Rp, jnp.float32(_M))):
            plsc.store_scatter(ref, [g_cnt], zf + tot, mask=lane0)
        NG[...] = g_cnt

        def zero_body(j, carry):
            for u in range(8):
                OUT[pl.ds(j * 128 + u * 16, 16)] = zf16
            return carry

        lax.fori_loop(0, (2 * _CH) // 8, zero_body, 0)
        g_scal = jnp.max(g_cnt)

        def fin_body(j, carry):
            base = j * 16
            g_i = iota + base
            valid = g_i < g_cnt
            d_s = Rs[pl.ds(base + 1, 16)] - Rs[pl.ds(base, 16)]
            d_e = Re[pl.ds(base + 1, 16)] - Re[pl.ds(base, 16)]
            d_sr = Rsr[pl.ds(base + 1, 16)] - Rsr[pl.ds(base, 16)]
            d_er = Rer[pl.ds(base + 1, 16)] - Rer[pl.ds(base, 16)]
            d_r = Rr[pl.ds(base + 1, 16)] - Rr[pl.ds(base, 16)]
            d_p = Rp[pl.ds(base + 1, 16)] - Rp[pl.ds(base, 16)]
            has1 = d_r > jnp.float32(0.5)
            num_s = jnp.where(has1, d_sr, d_s)
            num_e = jnp.where(has1, d_er, d_e)
            den = jnp.where(has1, d_r, jnp.maximum(d_p, jnp.float32(1)))
            os_ = jnp.where(valid, num_s / den, jnp.float32(0))
            oe_ = jnp.where(valid, num_e / den, jnp.float32(0))
            idx_s = (j >> 3) * 256 + (j & 7) * 16 + iota
            plsc.store_scatter(OUT, [idx_s], os_)
            plsc.store_scatter(OUT, [idx_s + 128], oe_)
            return carry

        lax.fori_loop(0, (g_scal + 15) // 16, fin_body, 0)

        pltpu.sync_copy(OUT, out_hbm.at[pl.ds(b * 2 * _M, 2 * _M)])
        pltpu.sync_copy(NG, ng_hbm.at[pl.ds(b * 16, 16)])


def kernel(frames_of_fragments_batch, frame_offsets_samples_batch):
    B, F, N, _ = frames_of_fragments_batch.shape
    M = F * N
    fr_flat = frames_of_fragments_batch.transpose(0, 2, 3, 1).reshape(B * M * 2)
    off_flat = frame_offsets_samples_batch.reshape(B * F)
    out_flat, ng_flat = _resolve_kernel(fr_flat, off_flat)
    resolved = (out_flat.reshape(B, M // 128, 2, 128)
                .transpose(0, 1, 3, 2).reshape(B, M, 2))
    num_groups = ng_flat.reshape(B, 16)[:, 0]
    return resolved, num_groups


# speedup vs baseline: 2.6558x; 1.0762x over previous
import functools

import jax
import jax.numpy as jnp
from jax import lax
from jax.experimental import pallas as pl
from jax.experimental.pallas import tpu as pltpu
from jax.experimental.pallas import tpu_sc as plsc

_B, _F, _N = 8, 128, 32
_M = _F * _N
_CH = _M // 16
_NGC = 64
_SIF = 16000
_COEF = _NGC / _SIF

_mesh = plsc.VectorSubcoreMesh(
    core_axis_name="c", subcore_axis_name="s", num_cores=2, num_subcores=16
)


@functools.partial(
    pl.kernel,
    out_type=(
        jax.ShapeDtypeStruct((_B * 2 * _M,), jnp.float32),
        jax.ShapeDtypeStruct((_B * 16,), jnp.int32),
    ),
    mesh=_mesh,
    compiler_params=pltpu.CompilerParams(needs_layout_passes=False),
    scratch_types=[
        pltpu.VMEM((2 * _M,), jnp.float32),
        pltpu.VMEM((_M // 4,), jnp.float32),
        pltpu.VMEM((_M // 4,), jnp.float32),
        pltpu.VMEM_SHARED((4, 2, _M), jnp.float32),
        pltpu.VMEM((_M,), jnp.float32),
        pltpu.VMEM((_M,), jnp.float32),
        pltpu.VMEM((_F,), jnp.int32),
        pltpu.VMEM((_M + 16,), jnp.float32),
        pltpu.VMEM((_M + 16,), jnp.float32),
        pltpu.VMEM((_M + 16,), jnp.float32),
        pltpu.VMEM((_M + 16,), jnp.float32),
        pltpu.VMEM((_M + 16,), jnp.float32),
        pltpu.VMEM((_M + 16,), jnp.float32),
        pltpu.VMEM((17,), jnp.float32),
        pltpu.VMEM((2 * _M,), jnp.float32),
        pltpu.VMEM((16,), jnp.int32),
    ],
)
def _resolve_kernel(fr_hbm, off_hbm, out_hbm, ng_hbm,
                    FRQ, SQ, EQ, SHD, S, E, OFF, Rs, Re, Rsr, Rer, Rr, Rp,
                    SH, OUT, NG):
    wid = lax.axis_index("c") * 16 + lax.axis_index("s")
    b = wid // 4
    q = wid % 4
    bl = b % 4

    iota = lax.iota(jnp.int32, 16)
    lane0 = iota == 0
    neg_inf = jnp.float32(jnp.finfo(jnp.float32).min)

    pltpu.sync_copy(fr_hbm.at[pl.ds(b * 2 * _M, 2 * _M)], FRQ)

    def tr_body(k, carry):
        bn = k // 8
        fb = (k // 4) % 2
        d0 = (k % 4) * 4
        for u in range(4):
            rot = (iota + (d0 + u)) & 15
            floc = fb * 16 + rot
            src = (bn * 16 + iota) * 256 + 32 * q + floc
            dst = floc * 32 + bn * 16 + iota
            plsc.store_scatter(SQ, [dst], plsc.load_gather(FRQ, [src]))
            plsc.store_scatter(EQ, [dst], plsc.load_gather(FRQ, [src + 128]))
        return carry

    lax.fori_loop(0, 16, tr_body, 0)

    def sort_body(f, carry):
        b0 = f * 32
        ak = SQ[pl.ds(b0, 16)]
        bk = SQ[pl.ds(b0 + 16, 16)]
        av = EQ[pl.ds(b0, 16)]
        bv = EQ[pl.ds(b0 + 16, 16)]
        ak, av = plsc.sort_key_val(ak, av)
        bk, bv = plsc.sort_key_val(bk, bv)
        rbk = lax.rev(bk, (0,))
        rbv = lax.rev(bv, (0,))
        ta = ak <= rbk
        lok = jnp.where(ta, ak, rbk)
        lov = jnp.where(ta, av, rbv)
        hik = jnp.where(ta, rbk, ak)
        hiv = jnp.where(ta, rbv, av)
        lok, lov = plsc.sort_key_val(lok, lov)
        hik, hiv = plsc.sort_key_val(hik, hiv)
        SQ[pl.ds(b0, 16)] = lok
        SQ[pl.ds(b0 + 16, 16)] = hik
        EQ[pl.ds(b0, 16)] = lov
        EQ[pl.ds(b0 + 16, 16)] = hiv
        return carry

    lax.fori_loop(0, _F // 4, sort_body, 0)

    pltpu.sync_copy(SQ, SHD.at[bl, 0, pl.ds((_M // 4) * q, _M // 4)])
    pltpu.sync_copy(EQ, SHD.at[bl, 1, pl.ds((_M // 4) * q, _M // 4)])
    plsc.subcore_barrier()

    @pl.when(q == 0)
    def _():
        pltpu.sync_copy(SHD.at[bl, 0], S)
        pltpu.sync_copy(SHD.at[bl, 1], E)
        pltpu.sync_copy(off_hbm.at[pl.ds(b * _F, _F)], OFF)

        SH[pl.ds(0, 16)] = jnp.full((16,), neg_inf, jnp.float32)
        coef = jnp.float32(_COEF)

        def scan_body(i, carry):
            c_m, c_s, c_e, c_sr, c_er, c_r, c_g = carry
            sv = S[pl.ds(i * 16, 16)]
            ev = E[pl.ds(i * 16, 16)]
            off = plsc.load_gather(
                OFF, [jnp.zeros((16,), jnp.int32) + (i // 2)]
            ).astype(jnp.float32)
            t1 = ((sv - off) * coef).astype(jnp.int32)
            t2 = ((ev - off) * coef).astype(jnp.int32)
            rv = jnp.where((t1 <= 0) | (t2 >= _NGC - 1),
                           jnp.float32(0), jnp.float32(1))
            cm = plsc.cummax(ev)
            lane15 = jnp.full((16,), 15, jnp.int32)
            SH[pl.ds(1, 16)] = cm
            shifted = SH[pl.ds(0, 16)]
            c_m_new = jnp.maximum(c_m, plsc.load_gather(SH, [lane15 + 1]))
            excl = jnp.maximum(shifted, c_m)
            flags = sv > excl
            gidx = c_g + plsc.cumsum(flags.astype(jnp.int32)) - 1

            srv = sv * rv
            erv = ev * rv
            new_c = []
            for ref, v, c in ((Rs, sv, c_s), (Re, ev, c_e), (Rsr, srv, c_sr),
                              (Rer, erv, c_er), (Rr, rv, c_r)):
                p_incl = c + plsc.cumsum(v)
                plsc.store_scatter(ref, [gidx], p_incl - v, mask=flags)
                SH[pl.ds(1, 16)] = p_incl
                new_c.append(plsc.load_gather(SH, [lane15 + 1]))
            pos = (iota + i * 16).astype(jnp.float32)
            plsc.store_scatter(Rp, [gidx], pos, mask=flags)

            return (c_m_new, new_c[0], new_c[1], new_c[2], new_c[3], new_c[4],
                    c_g + plsc.all_reduce_population_count(flags))

        zf16 = jnp.zeros((16,), jnp.float32)
        init = (jnp.full((16,), neg_inf, jnp.float32), zf16, zf16, zf16,
                zf16, zf16, jnp.zeros((16,), jnp.int32))
        (_, t_s, t_e, t_sr, t_er, t_r, g_cnt) = lax.fori_loop(
            0, _CH, scan_body, init)

        zf = jnp.zeros((16,), jnp.float32)
        for ref, tot in ((Rs, t_s), (Re, t_e), (Rsr, t_sr), (Rer, t_er),
                         (Rr, t_r), (Rp, jnp.float32(_M))):
            plsc.store_scatter(ref, [g_cnt], zf + tot, mask=lane0)
        NG[...] = g_cnt

        def zero_body(j, carry):
            for u in range(8):
                OUT[pl.ds(j * 128 + u * 16, 16)] = zf16
            return carry

        lax.fori_loop(0, (2 * _CH) // 8, zero_body, 0)
        g_scal = jnp.max(g_cnt)

        def fin_body(j, carry):
            base = j * 16
            g_i = iota + base
            valid = g_i < g_cnt
            d_s = Rs[pl.ds(base + 1, 16)] - Rs[pl.ds(base, 16)]
            d_e = Re[pl.ds(base + 1, 16)] - Re[pl.ds(base, 16)]
            d_sr = Rsr[pl.ds(base + 1, 16)] - Rsr[pl.ds(base, 16)]
            d_er = Rer[pl.ds(base + 1, 16)] - Rer[pl.ds(base, 16)]
            d_r = Rr[pl.ds(base + 1, 16)] - Rr[pl.ds(base, 16)]
            d_p = Rp[pl.ds(base + 1, 16)] - Rp[pl.ds(base, 16)]
            has1 = d_r > jnp.float32(0.5)
            num_s = jnp.where(has1, d_sr, d_s)
            num_e = jnp.where(has1, d_er, d_e)
            den = jnp.where(has1, d_r, jnp.maximum(d_p, jnp.float32(1)))
            os_ = jnp.where(valid, num_s / den, jnp.float32(0))
            oe_ = jnp.where(valid, num_e / den, jnp.float32(0))
            idx_s = (j >> 3) * 256 + (j & 7) * 16 + iota
            plsc.store_scatter(OUT, [idx_s], os_)
            plsc.store_scatter(OUT, [idx_s + 128], oe_)
            return carry

        lax.fori_loop(0, (g_scal + 15) // 16, fin_body, 0)

        pltpu.sync_copy(OUT, out_hbm.at[pl.ds(b * 2 * _M, 2 * _M)])
        pltpu.sync_copy(NG, ng_hbm.at[pl.ds(b * 16, 16)])


def kernel(frames_of_fragments_batch, frame_offsets_samples_batch):
    B, F, N, _ = frames_of_fragments_batch.shape
    M = F * N
    fr_flat = frames_of_fragments_batch.transpose(0, 2, 3, 1).reshape(B * M * 2)
    off_flat = frame_offsets_samples_batch.reshape(B * F)
    out_flat, ng_flat = _resolve_kernel(fr_flat, off_flat)
    resolved = (out_flat.reshape(B, M // 128, 2, 128)
                .transpose(0, 1, 3, 2).reshape(B, M, 2))
    num_groups = ng_flat.reshape(B, 16)[:, 0]
    return resolved, num_groups
